# baseline (speedup 1.0000x reference)
import functools

import jax
import jax.numpy as jnp
from jax import lax
from jax.experimental import pallas as pl
from jax.experimental.pallas import tpu as pltpu
from jax.experimental.pallas import tpu_sc as plsc

B = 4
C = 384
HW = 128 * 128
NCLS = 21
NBINS = 32
NC = 2
NS = 16
NW = NC * NS
CPW = C // NW
K = 4
L = 16

A_OFF = 0
S2_OFF = CPW * K * NBINS
CNT_OFF = S2_OFF + K * NBINS
ACC_SIZE = CNT_OFF + K * NBINS

STEPS = HW // L // K


def _sc_body(feat_hbm, lab_hbm, protot_hbm, out_hbm,
             lab_v, plane_v, acc_v, prot_v, outst_v, sem):
    del sem
    wid = lax.axis_index("s") * NC + lax.axis_index("c")
    ch0 = wid * CPW

    zeros = jnp.zeros((L,), jnp.float32)
    ones = jnp.full((L,), 1.0, jnp.float32)

    def zbody(i, _):
        acc_v[pl.ds(i * L, L)] = zeros
        return 0
    lax.fori_loop(0, ACC_SIZE // L, zbody, 0)

    for b in range(B):
        pltpu.sync_copy(lab_hbm.at[b], lab_v)

        @pl.when(wid == b)
        def _count():
            def cbody(j, _):
                for k in range(K):
                    lab = lab_v[pl.ds((j * K + k) * L, L)]
                    plsc.addupdate_scatter(
                        acc_v, [lab + (CNT_OFF + k * NBINS)], ones)
                return 0
            lax.fori_loop(0, STEPS, cbody, 0)

        def chbody(i, _):
            pltpu.sync_copy(feat_hbm.at[b, ch0 + i], plane_v)
            abase = i * (K * NBINS)

            def pbody(j, _):
                for k in range(K):
                    off = (j * K + k) * L
                    v = plane_v[pl.ds(off, L)]
                    lab = lab_v[pl.ds(off, L)]
                    plsc.addupdate_scatter(
                        acc_v, [lab + (abase + k * NBINS)], v)
                    plsc.addupdate_scatter(
                        acc_v, [lab + (S2_OFF + k * NBINS)], v * v)
                return 0
            lax.fori_loop(0, STEPS, pbody, 0)
            return 0
        lax.fori_loop(0, CPW, chbody, 0)

    pltpu.sync_copy(protot_hbm.at[pl.ds(ch0 * NBINS, CPW * NBINS)], prot_v)

    for h in range(NBINS // L):
        s2tot = zeros
        cnttot = zeros
        for k in range(K):
            s2tot = s2tot + acc_v[pl.ds(S2_OFF + k * NBINS + h * L, L)]
            cnttot = cnttot + acc_v[pl.ds(CNT_OFF + k * NBINS + h * L, L)]

        def dbody(i, carry):
            d, p2 = carry
            arow = zeros
            for k in range(K):
                arow = arow + acc_v[pl.ds(i * (K * NBINS) + k * NBINS + h * L, L)]
            p = prot_v[pl.ds(i * NBINS + h * L, L)]
            return (d + arow * p, p2 + p * p)
        d, p2 = lax.fori_loop(0, CPW, dbody, (zeros, zeros))

        outst_v[pl.ds(0 * NBINS + h * L, L)] = d
        outst_v[pl.ds(1 * NBINS + h * L, L)] = s2tot
        outst_v[pl.ds(2 * NBINS + h * L, L)] = cnttot
        outst_v[pl.ds(3 * NBINS + h * L, L)] = p2

    pltpu.sync_copy(outst_v, out_hbm.at[wid])


_sc_call = functools.partial(
    pl.kernel,
    out_type=jax.ShapeDtypeStruct((NW, 4 * NBINS), jnp.float32),
    mesh=plsc.VectorSubcoreMesh(core_axis_name="c", subcore_axis_name="s"),
    compiler_params=pltpu.CompilerParams(needs_layout_passes=False),
    scratch_types=[
        pltpu.VMEM((HW,), jnp.int32),
        pltpu.VMEM((HW,), jnp.float32),
        pltpu.VMEM((ACC_SIZE,), jnp.float32),
        pltpu.VMEM((CPW * NBINS,), jnp.float32),
        pltpu.VMEM((4 * NBINS,), jnp.float32),
        pltpu.SemaphoreType.DMA,
    ],
)(_sc_body)


def _fin_body(p_ref, o_ref):
    x = p_ref[...]
    s = jnp.sum(x, axis=0)
    d = s[0]
    s2 = s[1]
    cnt = s[2]
    p2 = s[3]
    present = cnt > 0.0
    denom = jnp.where(present, cnt * jnp.float32(C), jnp.float32(1.0))
    term = (s2 - 2.0 * d + cnt * p2) / denom
    loss = (jnp.sum(jnp.where(present, term, jnp.float32(0.0)))
            / jnp.sum(present.astype(jnp.float32)))
    o_ref[...] = jnp.reshape(loss, (1, 1))


_fin_call = pl.pallas_call(
    _fin_body,
    out_shape=jax.ShapeDtypeStruct((1, 1), jnp.float32),
)


def kernel(features, labels, prototypes):
    feat = features.reshape(B, C, HW)
    lab = labels.reshape(B, HW)
    protot = jnp.pad(prototypes.T, ((0, 0), (0, NBINS - NCLS))).reshape(C * NBINS)
    partials = _sc_call(feat, lab, protot)
    loss = _fin_call(partials.reshape(NW, 4, NBINS))
    return loss[0, 0]


# speedup vs baseline: 1.5941x; 1.5941x over previous
import functools

import jax
import jax.numpy as jnp
from jax import lax
from jax.experimental import pallas as pl
from jax.experimental.pallas import tpu as pltpu
from jax.experimental.pallas import tpu_sc as plsc

B = 4
C = 384
HW = 128 * 128
NCLS = 21
NBINS = 32
NC = 2
NS = 16
NW = NC * NS
CPW = C // NW
K = 4
L = 16

A_OFF = 0
S2_OFF = CPW * K * NBINS
CNT_OFF = S2_OFF + K * NBINS
ACC_SIZE = CNT_OFF + K * NBINS

STEPS = HW // L // K


def _sc_body(feat_hbm, lab_hbm, protot_hbm, out_hbm,
             lab_v, plane_v, acc_v, prot_v, outst_v, sem):
    del sem
    wid = lax.axis_index("s") * NC + lax.axis_index("c")
    ch0 = wid * CPW

    zeros = jnp.zeros((L,), jnp.float32)
    ones = jnp.full((L,), 1.0, jnp.float32)

    def zbody(i):
        acc_v[pl.ds(i * L, L)] = zeros
    plsc.parallel_loop(0, ACC_SIZE // L, 1, unroll=4)(zbody)

    for b in range(B):
        pltpu.sync_copy(lab_hbm.at[b], lab_v)

        @pl.when(wid == b)
        def _count():
            def cbody(j):
                lab = lab_v[pl.ds(j * L, L)]
                k = jnp.bitwise_and(j, K - 1)
                plsc.addupdate_scatter(
                    acc_v, [lab + (CNT_OFF + k * NBINS)], ones)
            plsc.parallel_loop(0, HW // L, 1, unroll=K)(cbody)

        def chbody(i, _):
            pltpu.sync_copy(feat_hbm.at[b, ch0 + i], plane_v)
            abase = i * (K * NBINS)

            def pstep(j):
                off = j * L
                v = plane_v[pl.ds(off, L)]
                lab = lab_v[pl.ds(off, L)]
                k = jnp.bitwise_and(j, K - 1)
                plsc.addupdate_scatter(
                    acc_v, [lab + (abase + k * NBINS)], v)
                plsc.addupdate_scatter(
                    acc_v, [lab + (S2_OFF + k * NBINS)], v * v)
            plsc.parallel_loop(0, HW // L, 1, unroll=2 * K)(pstep)
            return 0
        lax.fori_loop(0, CPW, chbody, 0)

    pltpu.sync_copy(protot_hbm.at[pl.ds(ch0 * NBINS, CPW * NBINS)], prot_v)

    for h in range(NBINS // L):
        s2tot = zeros
        cnttot = zeros
        for k in range(K):
            s2tot = s2tot + acc_v[pl.ds(S2_OFF + k * NBINS + h * L, L)]
            cnttot = cnttot + acc_v[pl.ds(CNT_OFF + k * NBINS + h * L, L)]

        def dbody(i, carry):
            d, p2 = carry
            arow = zeros
            for k in range(K):
                arow = arow + acc_v[pl.ds(i * (K * NBINS) + k * NBINS + h * L, L)]
            p = prot_v[pl.ds(i * NBINS + h * L, L)]
            return (d + arow * p, p2 + p * p)
        d, p2 = lax.fori_loop(0, CPW, dbody, (zeros, zeros))

        outst_v[pl.ds(0 * NBINS + h * L, L)] = d
        outst_v[pl.ds(1 * NBINS + h * L, L)] = s2tot
        outst_v[pl.ds(2 * NBINS + h * L, L)] = cnttot
        outst_v[pl.ds(3 * NBINS + h * L, L)] = p2

    pltpu.sync_copy(outst_v, out_hbm.at[wid])


_sc_call = functools.partial(
    pl.kernel,
    out_type=jax.ShapeDtypeStruct((NW, 4 * NBINS), jnp.float32),
    mesh=plsc.VectorSubcoreMesh(core_axis_name="c", subcore_axis_name="s"),
    compiler_params=pltpu.CompilerParams(needs_layout_passes=False),
    scratch_types=[
        pltpu.VMEM((HW,), jnp.int32),
        pltpu.VMEM((HW,), jnp.float32),
        pltpu.VMEM((ACC_SIZE,), jnp.float32),
        pltpu.VMEM((CPW * NBINS,), jnp.float32),
        pltpu.VMEM((4 * NBINS,), jnp.float32),
        pltpu.SemaphoreType.DMA,
    ],
)(_sc_body)


def _fin_body(p_ref, o_ref):
    x = p_ref[...]
    s = jnp.sum(x, axis=0)
    d = s[0]
    s2 = s[1]
    cnt = s[2]
    p2 = s[3]
    present = cnt > 0.0
    denom = jnp.where(present, cnt * jnp.float32(C), jnp.float32(1.0))
    term = (s2 - 2.0 * d + cnt * p2) / denom
    loss = (jnp.sum(jnp.where(present, term, jnp.float32(0.0)))
            / jnp.sum(present.astype(jnp.float32)))
    o_ref[...] = jnp.reshape(loss, (1, 1))


_fin_call = pl.pallas_call(
    _fin_body,
    out_shape=jax.ShapeDtypeStruct((1, 1), jnp.float32),
)


def kernel(features, labels, prototypes):
    feat = features.reshape(B, C, HW)
    lab = labels.reshape(B, HW)
    protot = jnp.pad(prototypes.T, ((0, 0), (0, NBINS - NCLS))).reshape(C * NBINS)
    partials = _sc_call(feat, lab, protot)
    loss = _fin_call(partials.reshape(NW, 4, NBINS))
    return loss[0, 0]


# speedup vs baseline: 2.0604x; 1.2925x over previous
import functools

import jax
import jax.numpy as jnp
from jax import lax
from jax.experimental import pallas as pl
from jax.experimental.pallas import tpu as pltpu
from jax.experimental.pallas import tpu_sc as plsc

B = 4
C = 384
HW = 128 * 128
NCLS = 21
NBINS = 32
NC = 2
NS = 16
NW = NC * NS
CPW = C // NW
K = 4
L = 16

A_OFF = 0
S2_OFF = CPW * K * NBINS
CNT_OFF = S2_OFF + K * NBINS
ACC_SIZE = CNT_OFF + K * NBINS

STEPS = HW // L // K


NPL = B * CPW


def _sc_body(feat_hbm, lab_hbm, protot_hbm, out_hbm,
             lab_v, plane_v, acc_v, prot_v, outst_v, sem0, sem1):
    wid = lax.axis_index("s") * NC + lax.axis_index("c")
    ch0 = wid * CPW
    sems = (sem0, sem1)

    zeros = jnp.zeros((L,), jnp.float32)
    ones = jnp.full((L,), 1.0, jnp.float32)

    def zbody(i):
        acc_v[pl.ds(i * L, L)] = zeros
    plsc.parallel_loop(0, ACC_SIZE // L, 1, unroll=4)(zbody)

    pltpu.sync_copy(lab_hbm, lab_v)

    def _start(t, par):
        b = jnp.bitwise_and(t, B - 1)
        ch = ch0 + lax.shift_right_logical(t, 2)
        g = b * C + ch
        pltpu.async_copy(
            feat_hbm.at[g], plane_v.at[pl.ds(par * HW, HW)], sems[par])

    _start(0, 0)

    @pl.when(wid < B)
    def _count():
        cbase = wid * HW

        def cbody(j):
            lab = lab_v[pl.ds(cbase + j * L, L)]
            k = jnp.bitwise_and(j, K - 1)
            plsc.addupdate_scatter(
                acc_v, [lab + (CNT_OFF + k * NBINS)], ones)
        plsc.parallel_loop(0, HW // L, 1, unroll=K)(cbody)

    def _compute(t, par):
        lab_base = jnp.bitwise_and(t, B - 1) * HW
        abase = lax.shift_right_logical(t, 2) * (K * NBINS)
        pbase = par * HW

        def pstep(j):
            off = j * L
            v = plane_v[pl.ds(pbase + off, L)]
            lab = lab_v[pl.ds(lab_base + off, L)]
            k = jnp.bitwise_and(j, K - 1)
            plsc.addupdate_scatter(
                acc_v, [lab + (abase + k * NBINS)], v)
            plsc.addupdate_scatter(
                acc_v, [lab + (S2_OFF + k * NBINS)], v * v)
        plsc.parallel_loop(0, HW // L, 1, unroll=2 * K)(pstep)

    def pairbody(tp, _):
        for par in range(2):
            t = tp * 2 + par
            pltpu.make_async_copy(
                feat_hbm.at[0], plane_v.at[pl.ds(par * HW, HW)],
                sems[par]).wait()

            @pl.when(t + 1 < NPL)
            def _prefetch():
                _start(t + 1, 1 - par)

            _compute(t, par)
        return 0
    lax.fori_loop(0, NPL // 2, pairbody, 0)

    pltpu.sync_copy(protot_hbm.at[pl.ds(ch0 * NBINS, CPW * NBINS)], prot_v)

    for h in range(NBINS // L):
        s2tot = zeros
        cnttot = zeros
        for k in range(K):
            s2tot = s2tot + acc_v[pl.ds(S2_OFF + k * NBINS + h * L, L)]
            cnttot = cnttot + acc_v[pl.ds(CNT_OFF + k * NBINS + h * L, L)]

        def dbody(i, carry):
            d, p2 = carry
            arow = zeros
            for k in range(K):
                arow = arow + acc_v[pl.ds(i * (K * NBINS) + k * NBINS + h * L, L)]
            p = prot_v[pl.ds(i * NBINS + h * L, L)]
            return (d + arow * p, p2 + p * p)
        d, p2 = lax.fori_loop(0, CPW, dbody, (zeros, zeros))

        outst_v[pl.ds(0 * NBINS + h * L, L)] = d
        outst_v[pl.ds(1 * NBINS + h * L, L)] = s2tot
        outst_v[pl.ds(2 * NBINS + h * L, L)] = cnttot
        outst_v[pl.ds(3 * NBINS + h * L, L)] = p2

    pltpu.sync_copy(outst_v, out_hbm.at[wid])


_sc_call = functools.partial(
    pl.kernel,
    out_type=jax.ShapeDtypeStruct((NW, 4 * NBINS), jnp.float32),
    mesh=plsc.VectorSubcoreMesh(core_axis_name="c", subcore_axis_name="s"),
    compiler_params=pltpu.CompilerParams(needs_layout_passes=False),
    scratch_types=[
        pltpu.VMEM((B * HW,), jnp.int32),
        pltpu.VMEM((2 * HW,), jnp.float32),
        pltpu.VMEM((ACC_SIZE,), jnp.float32),
        pltpu.VMEM((CPW * NBINS,), jnp.float32),
        pltpu.VMEM((4 * NBINS,), jnp.float32),
        pltpu.SemaphoreType.DMA,
        pltpu.SemaphoreType.DMA,
    ],
)(_sc_body)


def _fin_body(p_ref, o_ref):
    x = p_ref[...]
    s = jnp.sum(x, axis=0)
    d = s[0]
    s2 = s[1]
    cnt = s[2]
    p2 = s[3]
    present = cnt > 0.0
    denom = jnp.where(present, cnt * jnp.float32(C), jnp.float32(1.0))
    term = (s2 - 2.0 * d + cnt * p2) / denom
    loss = (jnp.sum(jnp.where(present, term, jnp.float32(0.0)))
            / jnp.sum(present.astype(jnp.float32)))
    o_ref[...] = jnp.reshape(loss, (1, 1))


_fin_call = pl.pallas_call(
    _fin_body,
    out_shape=jax.ShapeDtypeStruct((1, 1), jnp.float32),
)


def kernel(features, labels, prototypes):
    feat = features.reshape(B * C, HW)
    lab = labels.reshape(B * HW)
    protot = jnp.pad(prototypes.T, ((0, 0), (0, NBINS - NCLS))).reshape(C * NBINS)
    partials = _sc_call(feat, lab, protot)
    loss = _fin_call(partials.reshape(NW, 4, NBINS))
    return loss[0, 0]


# speedup vs baseline: 2.6054x; 1.2645x over previous
import functools

import jax
import jax.numpy as jnp
from jax import lax
from jax.experimental import pallas as pl
from jax.experimental.pallas import tpu as pltpu
from jax.experimental.pallas import tpu_sc as plsc

B = 4
C = 384
HW = 128 * 128
NCLS = 21
NBINS = 32
NC = 2
NS = 16
NW = NC * NS
CPW = C // NW
K = 4
L = 16

A_OFF = 0
S2_OFF = CPW * K * NBINS
CNT_OFF = S2_OFF + K * NBINS
ACC_SIZE = CNT_OFF + K * NBINS

STEPS = HW // L // K


NPL = B * CPW


def _sc_body(feat_hbm, lab_hbm, protot_hbm, out_hbm,
             lab_v, plane_v, acc_v, prot_v, outst_v, sem0, sem1):
    wid = lax.axis_index("s") * NC + lax.axis_index("c")
    ch0 = wid * CPW
    sems = (sem0, sem1)

    zeros = jnp.zeros((L,), jnp.float32)
    ones = jnp.full((L,), 1.0, jnp.float32)

    def zbody(i):
        acc_v[pl.ds(i * L, L)] = zeros
    plsc.parallel_loop(0, ACC_SIZE // L, 1, unroll=4)(zbody)

    pltpu.sync_copy(lab_hbm, lab_v)

    def _start(t, par):
        b = jnp.bitwise_and(t, B - 1)
        ch = ch0 + lax.shift_right_logical(t, 2)
        pltpu.async_copy(feat_hbm.at[b, ch], plane_v.at[par], sems[par])

    _start(0, 0)

    @pl.when(wid < B)
    def _count():
        def cbody(j):
            r = lax.shift_right_logical(j, 3)
            cc = jnp.bitwise_and(j, 7) * L
            lab = lab_v[wid, r, pl.ds(cc, L)]
            k = jnp.bitwise_and(j, K - 1)
            plsc.addupdate_scatter(
                acc_v, [lab + (CNT_OFF + k * NBINS)], ones)
        plsc.parallel_loop(0, HW // L, 1, unroll=K)(cbody)

    def _compute(t, par):
        b = jnp.bitwise_and(t, B - 1)
        abase = lax.shift_right_logical(t, 2) * (K * NBINS)

        def pstep(j):
            r = lax.shift_right_logical(j, 3)
            cc = jnp.bitwise_and(j, 7) * L
            v = plane_v[par, r, pl.ds(cc, L)]
            lab = lab_v[b, r, pl.ds(cc, L)]
            k = jnp.bitwise_and(j, K - 1)
            plsc.addupdate_scatter(
                acc_v, [lab + (abase + k * NBINS)], v)
            plsc.addupdate_scatter(
                acc_v, [lab + (S2_OFF + k * NBINS)], v * v)
        plsc.parallel_loop(0, HW // L, 1, unroll=2 * K)(pstep)

    def pairbody(tp, _):
        for par in range(2):
            t = tp * 2 + par
            pltpu.make_async_copy(
                feat_hbm.at[0, 0], plane_v.at[par], sems[par]).wait()

            @pl.when(t + 1 < NPL)
            def _prefetch():
                _start(t + 1, 1 - par)

            _compute(t, par)
        return 0
    lax.fori_loop(0, NPL // 2, pairbody, 0)

    pltpu.sync_copy(protot_hbm.at[pl.ds(ch0 * NBINS, CPW * NBINS)], prot_v)

    for h in range(NBINS // L):
        s2tot = zeros
        cnttot = zeros
        for k in range(K):
            s2tot = s2tot + acc_v[pl.ds(S2_OFF + k * NBINS + h * L, L)]
            cnttot = cnttot + acc_v[pl.ds(CNT_OFF + k * NBINS + h * L, L)]

        def dbody(i, carry):
            d, p2 = carry
            arow = zeros
            for k in range(K):
                arow = arow + acc_v[pl.ds(i * (K * NBINS) + k * NBINS + h * L, L)]
            p = prot_v[pl.ds(i * NBINS + h * L, L)]
            return (d + arow * p, p2 + p * p)
        d, p2 = lax.fori_loop(0, CPW, dbody, (zeros, zeros))

        outst_v[pl.ds(0 * NBINS + h * L, L)] = d
        outst_v[pl.ds(1 * NBINS + h * L, L)] = s2tot
        outst_v[pl.ds(2 * NBINS + h * L, L)] = cnttot
        outst_v[pl.ds(3 * NBINS + h * L, L)] = p2

    pltpu.sync_copy(outst_v, out_hbm.at[wid])


_sc_call = functools.partial(
    pl.kernel,
    out_type=jax.ShapeDtypeStruct((NW, 4 * NBINS), jnp.float32),
    mesh=plsc.VectorSubcoreMesh(core_axis_name="c", subcore_axis_name="s"),
    compiler_params=pltpu.CompilerParams(needs_layout_passes=False),
    scratch_types=[
        pltpu.VMEM((B, 128, 128), jnp.int32),
        pltpu.VMEM((2, 128, 128), jnp.float32),
        pltpu.VMEM((ACC_SIZE,), jnp.float32),
        pltpu.VMEM((CPW * NBINS,), jnp.float32),
        pltpu.VMEM((4 * NBINS,), jnp.float32),
        pltpu.SemaphoreType.DMA,
        pltpu.SemaphoreType.DMA,
    ],
)(_sc_body)


def _fin_body(p_ref, o_ref):
    x = p_ref[...]
    s = jnp.sum(x, axis=0)
    d = s[0]
    s2 = s[1]
    cnt = s[2]
    p2 = s[3]
    present = cnt > 0.0
    denom = jnp.where(present, cnt * jnp.float32(C), jnp.float32(1.0))
    term = (s2 - 2.0 * d + cnt * p2) / denom
    loss = (jnp.sum(jnp.where(present, term, jnp.float32(0.0)))
            / jnp.sum(present.astype(jnp.float32)))
    o_ref[...] = jnp.reshape(loss, (1, 1))


_fin_call = pl.pallas_call(
    _fin_body,
    out_shape=jax.ShapeDtypeStruct((1, 1), jnp.float32),
)


def kernel(features, labels, prototypes):
    protot = jnp.pad(prototypes.T, ((0, 0), (0, NBINS - NCLS))).reshape(C * NBINS)
    partials = _sc_call(features, labels, protot)
    loss = _fin_call(partials.reshape(NW, 4, NBINS))
    return loss[0, 0]


# speedup vs baseline: 9.4753x; 3.6369x over previous
import functools

import jax
import jax.numpy as jnp
from jax import lax
from jax.experimental import pallas as pl
from jax.experimental.pallas import tpu as pltpu
from jax.experimental.pallas import tpu_sc as plsc

B = 4
C = 384
HW = 128 * 128
NCLS = 21
NBINS = 32
NC = 2
NS = 16
NW = NC * NS
K = 4
L = 16

ROWS_PER_TILE = 16
PIX = ROWS_PER_TILE * 128
N_TILES = 128 // ROWS_PER_TILE

D_OFF = 0
S2_OFF = K * NBINS
CNT_OFF = 2 * K * NBINS
ACC_SIZE = 3 * K * NBINS

CHUNK_ROWS = 128 // (NW // B)


def _tc_main_body(prot_ref, feat_ref, lab_ref, d_ref, s_ref):
    x = feat_ref[0].reshape(C, PIX)
    p = prot_ref[...]
    dall = jnp.dot(p, x, preferred_element_type=jnp.float32)
    lab = lab_ref[0].reshape(1, PIX)
    cls = lax.broadcasted_iota(jnp.int32, (NBINS, PIX), 0)
    dsel = jnp.sum(jnp.where(cls == lab, dall, jnp.float32(0.0)), axis=0)
    ssq = jnp.sum(x * x, axis=0)
    d_ref[0] = dsel.reshape(ROWS_PER_TILE, 128)
    s_ref[0] = ssq.reshape(ROWS_PER_TILE, 128)


_tc_main = pl.pallas_call(
    _tc_main_body,
    grid=(B, N_TILES),
    in_specs=[
        pl.BlockSpec((NBINS, C), lambda b, j: (0, 0)),
        pl.BlockSpec((1, C, ROWS_PER_TILE, 128), lambda b, j: (b, 0, j, 0)),
        pl.BlockSpec((1, ROWS_PER_TILE, 128), lambda b, j: (b, j, 0)),
    ],
    out_specs=[
        pl.BlockSpec((1, ROWS_PER_TILE, 128), lambda b, j: (b, j, 0)),
        pl.BlockSpec((1, ROWS_PER_TILE, 128), lambda b, j: (b, j, 0)),
    ],
    out_shape=[
        jax.ShapeDtypeStruct((B, 128, 128), jnp.float32),
        jax.ShapeDtypeStruct((B, 128, 128), jnp.float32),
    ],
)


def _seg_body(d_hbm, s_hbm, lab_hbm, out_hbm,
              dv, sv, labv, acc_v, outst_v, sem0, sem1, sem2):
    wid = lax.axis_index("s") * NC + lax.axis_index("c")
    b = jnp.bitwise_and(wid, B - 1)
    ro = lax.shift_right_logical(wid, 2) * CHUNK_ROWS

    zeros = jnp.zeros((L,), jnp.float32)
    ones = jnp.full((L,), 1.0, jnp.float32)

    cd = pltpu.async_copy(d_hbm.at[b, pl.ds(ro, CHUNK_ROWS)], dv, sem0)
    cs = pltpu.async_copy(s_hbm.at[b, pl.ds(ro, CHUNK_ROWS)], sv, sem1)
    cl_ = pltpu.async_copy(lab_hbm.at[b, pl.ds(ro, CHUNK_ROWS)], labv, sem2)

    def zbody(i):
        acc_v[pl.ds(i * L, L)] = zeros
    plsc.parallel_loop(0, ACC_SIZE // L, 1, unroll=4)(zbody)

    cd.wait()
    cs.wait()
    cl_.wait()

    def pstep(j):
        r = lax.shift_right_logical(j, 3)
        cc = jnp.bitwise_and(j, 7) * L
        dd = dv[r, pl.ds(cc, L)]
        ss = sv[r, pl.ds(cc, L)]
        lab = labv[r, pl.ds(cc, L)]
        k = jnp.bitwise_and(j, K - 1)
        plsc.addupdate_scatter(acc_v, [lab + (D_OFF + k * NBINS)], dd)
        plsc.addupdate_scatter(acc_v, [lab + (S2_OFF + k * NBINS)], ss)
        plsc.addupdate_scatter(acc_v, [lab + (CNT_OFF + k * NBINS)], ones)
    plsc.parallel_loop(0, (CHUNK_ROWS * 128) // L, 1, unroll=4)(pstep)

    for q in range(3):
        for h in range(NBINS // L):
            tot = zeros
            for k in range(K):
                tot = tot + acc_v[pl.ds(q * K * NBINS + k * NBINS + h * L, L)]
            outst_v[pl.ds(q * NBINS + h * L, L)] = tot
    for h in range(NBINS // L):
        outst_v[pl.ds(3 * NBINS + h * L, L)] = zeros

    pltpu.sync_copy(outst_v, out_hbm.at[wid])


_seg_call = functools.partial(
    pl.kernel,
    out_type=jax.ShapeDtypeStruct((NW, 4 * NBINS), jnp.float32),
    mesh=plsc.VectorSubcoreMesh(core_axis_name="c", subcore_axis_name="s"),
    compiler_params=pltpu.CompilerParams(needs_layout_passes=False),
    scratch_types=[
        pltpu.VMEM((CHUNK_ROWS, 128), jnp.float32),
        pltpu.VMEM((CHUNK_ROWS, 128), jnp.float32),
        pltpu.VMEM((CHUNK_ROWS, 128), jnp.int32),
        pltpu.VMEM((ACC_SIZE,), jnp.float32),
        pltpu.VMEM((4 * NBINS,), jnp.float32),
        pltpu.SemaphoreType.DMA,
        pltpu.SemaphoreType.DMA,
        pltpu.SemaphoreType.DMA,
    ],
)(_seg_body)


def _fin_body(p_ref, prot_ref, o_ref):
    x = p_ref[...]
    s = jnp.sum(x, axis=0)
    d = s[0]
    s2 = s[1]
    cnt = s[2]
    pp = prot_ref[...]
    p2 = jnp.sum(pp * pp, axis=1)
    present = cnt > 0.0
    denom = jnp.where(present, cnt * jnp.float32(C), jnp.float32(1.0))
    term = (s2 - 2.0 * d + cnt * p2) / denom
    loss = (jnp.sum(jnp.where(present, term, jnp.float32(0.0)))
            / jnp.sum(present.astype(jnp.float32)))
    o_ref[...] = jnp.reshape(loss, (1, 1))


_fin_call = pl.pallas_call(
    _fin_body,
    out_shape=jax.ShapeDtypeStruct((1, 1), jnp.float32),
)


def kernel(features, labels, prototypes):
    protot = jnp.pad(prototypes, ((0, NBINS - NCLS), (0, 0)))
    d, s = _tc_main(protot, features, labels)
    partials = _seg_call(d, s, labels)
    loss = _fin_call(partials.reshape(NW, 4, NBINS), protot)
    return loss[0, 0]
